# baseline (speedup 1.0000x reference)
import functools

import jax
import jax.numpy as jnp
from jax import lax
from jax.experimental import pallas as pl
from jax.experimental.pallas import tpu as pltpu
from jax.experimental.pallas import tpu_sc as plsc

_N = 10000
_C = 64
_HID = 64
_F_IN = 128
_K = 10
_ALPHA = 0.1

_N_TILES = 32
_N_PAD = 10240
_ROWS_PER_TILE = _N_PAD // 16
_CH = 128


_SLAB = 1


def _make_deg_sweep(n_slabs):
  mesh = plsc.VectorSubcoreMesh(core_axis_name="c", subcore_axis_name="s")

  @functools.partial(
      pl.kernel,
      mesh=mesh,
      compiler_params=pltpu.CompilerParams(use_tc_tiling_on_sc=False),
      out_type=jax.ShapeDtypeStruct((2, _N_PAD, 16), jnp.float32),
      scratch_types=[
          pltpu.VMEM_SHARED((_N_PAD, 16), jnp.float32),
          pltpu.VMEM((n_slabs, _SLAB * _CH), jnp.int32),
          pltpu.VMEM((_CH, 16), jnp.float32),
          pltpu.VMEM((_CH, 16), jnp.float32),
      ],
  )
  def dsweep(dst_hbm, zeros16_hbm, ones16_hbm, dega_hbm,
             accd_sh, didx_v, ones_v, zbuf_v):
    cid = lax.axis_index("c")
    sid = lax.axis_index("s")
    wid = cid * 16 + sid
    row0 = sid * _ROWS_PER_TILE

    pltpu.sync_copy(zeros16_hbm, zbuf_v)
    for b in range(_ROWS_PER_TILE // _CH):
      pltpu.sync_copy(zbuf_v, accd_sh.at[pl.ds(row0 + b * _CH, _CH)])
    pltpu.sync_copy(ones16_hbm, ones_v)
    pltpu.sync_copy(dst_hbm.at[wid], didx_v)
    plsc.subcore_barrier()

    def body(j, carry):
      pltpu.sync_copy(ones_v, accd_sh.at[didx_v.at[j]], add=True)
      return carry

    lax.fori_loop(0, n_slabs, body, 0)
    plsc.subcore_barrier()

    for b in range(_ROWS_PER_TILE // _CH):
      pltpu.sync_copy(accd_sh.at[pl.ds(row0 + b * _CH, _CH)], zbuf_v)
      pltpu.sync_copy(zbuf_v, dega_hbm.at[cid, pl.ds(row0 + b * _CH, _CH)])

  return dsweep


def _make_prop_sweep(n_slabs):
  mesh = plsc.VectorSubcoreMesh(core_axis_name="c", subcore_axis_name="s")

  @functools.partial(
      pl.kernel,
      mesh=mesh,
      compiler_params=pltpu.CompilerParams(use_tc_tiling_on_sc=False),
      out_type=jax.ShapeDtypeStruct((2, _N_PAD, _C), jnp.float32),
      scratch_types=[
          pltpu.VMEM_SHARED((_N_PAD, _C), jnp.float32),
          pltpu.VMEM_SHARED((_N_PAD, _C), jnp.float32),
          pltpu.VMEM((n_slabs, _SLAB * _CH), jnp.int32),
          pltpu.VMEM((n_slabs, _SLAB * _CH), jnp.int32),
          pltpu.VMEM((_SLAB * _CH, _C), jnp.float32),
          pltpu.VMEM((_CH, _C), jnp.float32),
          pltpu.VMEM((_CH, _C), jnp.float32),
          pltpu.VMEM((_CH, 16), jnp.float32),
          pltpu.SemaphoreType.DMA,
          pltpu.SemaphoreType.DMA,
      ],
  )
  def msweep(agg_in_hbm, c_hbm, bh_hbm, src_hbm, dst_hbm, zeros_hbm,
             agg_hbm, acc_sh, u_sh, sidx_v, didx_v, rows_v,
             a0_v, a1_v, c_v, sem, isem):
    cid = lax.axis_index("c")
    sid = lax.axis_index("s")
    wid = cid * 16 + sid
    row0 = sid * _ROWS_PER_TILE

    bg = [pltpu.make_async_copy(src_hbm.at[wid], sidx_v, isem),
          pltpu.make_async_copy(dst_hbm.at[wid], didx_v, isem)]
    for b in range(_ROWS_PER_TILE // _CH):
      bg.append(pltpu.make_async_copy(
          zeros_hbm, acc_sh.at[pl.ds(row0 + b * _CH, _CH)], isem))
    for g in bg:
      g.start()

    for b in range(_ROWS_PER_TILE // _CH):
      r0 = row0 + b * _CH
      g0 = pltpu.make_async_copy(agg_in_hbm.at[0, pl.ds(r0, _CH)], a0_v, sem)
      g1 = pltpu.make_async_copy(agg_in_hbm.at[1, pl.ds(r0, _CH)], a1_v, sem)
      g2 = pltpu.make_async_copy(c_hbm.at[pl.ds(r0, _CH), pl.ds(0, 16)],
                                 c_v, sem)
      g3 = pltpu.make_async_copy(bh_hbm.at[pl.ds(r0, _CH)], rows_v, sem)
      g0.start()
      g1.start()
      g2.start()
      g3.start()
      g0.wait()
      g1.wait()
      g2.wait()
      g3.wait()

      def rowbody(i, carry):
        for r in (2 * i, 2 * i + 1):
          cc = c_v[r, pl.ds(0, 16)]
          for cg in range(_C // 16):
            s = pl.ds(cg * 16, 16)
            a0_v[r, s] = cc * (a0_v[r, s] + a1_v[r, s]) + rows_v[r, s]
        return carry

      lax.fori_loop(0, _CH // 2, rowbody, 0)
      pltpu.sync_copy(a0_v, u_sh.at[pl.ds(r0, _CH)])
    for g in bg:
      g.wait()
    plsc.subcore_barrier()

    def body(j, carry):
      pltpu.async_copy(u_sh.at[sidx_v.at[j]], rows_v, sem).wait()
      pltpu.sync_copy(rows_v, acc_sh.at[didx_v.at[j]], add=True)
      return carry

    lax.fori_loop(0, n_slabs, body, 0)
    plsc.subcore_barrier()

    bufs = (rows_v, a0_v)
    outs = []
    for b in range(_ROWS_PER_TILE // _CH):
      buf = bufs[b % 2]
      if b >= 2:
        outs[b - 2].wait()
      pltpu.sync_copy(acc_sh.at[pl.ds(row0 + b * _CH, _CH)], buf)
      o = pltpu.make_async_copy(
          buf, agg_hbm.at[cid, pl.ds(row0 + b * _CH, _CH)], sem)
      o.start()
      outs.append(o)
    outs[-2].wait()
    outs[-1].wait()

  return msweep


def _prep_body(x_ref, w1_ref, b1_ref, w2_ref, b2_ref, dega_ref,
               aggi_ref, c_ref, bh_ref, sd_ref):
  deg = dega_ref[0, :, 0:1] + dega_ref[1, :, 0:1]
  rows = lax.broadcasted_iota(jnp.int32, (_N_PAD, 1), 0)
  mask = rows < _N
  dinv = jnp.where(mask, lax.rsqrt(jnp.maximum(deg, 1e-12)), 0.0)
  h = jax.nn.relu(
      jnp.dot(x_ref[...], w1_ref[...], preferred_element_type=jnp.float32)
      + b1_ref[...])
  h = jnp.dot(h, w2_ref[...], preferred_element_type=jnp.float32) + b2_ref[...]
  sd = jnp.where(mask, 1.0 / jnp.where(mask, dinv, 1.0), 0.0)
  aggi_ref[0] = jnp.broadcast_to(sd, (_N_PAD, _C)) * h
  aggi_ref[1] = jnp.zeros((_N_PAD, _C), jnp.float32)
  c_ref[...] = jnp.broadcast_to((1.0 - _ALPHA) * dinv * dinv, (_N_PAD, _C))
  bh_ref[...] = _ALPHA * jnp.broadcast_to(dinv, (_N_PAD, _C)) * h
  sd_ref[...] = jnp.broadcast_to(sd, (_N_PAD, _C))


_prep_call = pl.pallas_call(
    _prep_body,
    out_shape=[
        jax.ShapeDtypeStruct((2, _N_PAD, _C), jnp.float32),
        jax.ShapeDtypeStruct((_N_PAD, _C), jnp.float32),
        jax.ShapeDtypeStruct((_N_PAD, _C), jnp.float32),
        jax.ShapeDtypeStruct((_N_PAD, _C), jnp.float32),
    ],
)


def _final_body(agg_ref, c_ref, bh_ref, sd_ref, out_ref):
  u = c_ref[...] * (agg_ref[0] + agg_ref[1]) + bh_ref[...]
  z = (u * sd_ref[...])[:_N]
  m = jnp.max(z, axis=1, keepdims=True)
  shifted = z - m
  out_ref[...] = shifted - jnp.log(
      jnp.sum(jnp.exp(shifted), axis=1, keepdims=True))


_final_call = pl.pallas_call(
    _final_body,
    out_shape=jax.ShapeDtypeStruct((_N, _C), jnp.float32),
)


def kernel(x, edge_index, W1, b1, W2, b2):
  e = edge_index.shape[1]
  e_full = e + _N
  n_slabs = -(-e_full // (_N_TILES * _SLAB * _CH))
  e_pad = _N_TILES * n_slabs * _SLAB * _CH

  src = edge_index[0]
  dst = edge_index[1]
  loop = jnp.arange(_N, dtype=jnp.int32)
  pad = jnp.full((e_pad - e_full,), _N, dtype=jnp.int32)
  src_w = jnp.concatenate([src, loop, pad]).reshape(
      _N_TILES, n_slabs, _SLAB * _CH)
  dst_w = jnp.concatenate([dst, loop, pad]).reshape(
      _N_TILES, n_slabs, _SLAB * _CH)

  x_pad = jnp.pad(x, ((0, _N_PAD - _N), (0, 0)))
  zeros_tile = jnp.zeros((_CH, _C), jnp.float32)
  zeros16 = jnp.zeros((_CH, 16), jnp.float32)
  ones16 = jnp.ones((_CH, 16), jnp.float32)

  msweep = _make_prop_sweep(n_slabs)
  dsweep = _make_deg_sweep(n_slabs)

  dega = dsweep(dst_w, zeros16, ones16)
  agg, c, bh, sd = _prep_call(x_pad, W1, b1.reshape(1, _HID), W2,
                              b2.reshape(1, _C), dega)
  for _ in range(_K):
    agg = msweep(agg, c, bh, src_w, dst_w, zeros_tile)
  return _final_call(agg, c, bh, sd)


# speedup vs baseline: 1.0286x; 1.0286x over previous
import functools

import jax
import jax.numpy as jnp
from jax import lax
from jax.experimental import pallas as pl
from jax.experimental.pallas import tpu as pltpu
from jax.experimental.pallas import tpu_sc as plsc

_N = 10000
_C = 64
_HID = 64
_F_IN = 128
_K = 10
_ALPHA = 0.1

_N_TILES = 32
_N_PAD = 10240
_ROWS_PER_TILE = _N_PAD // 16
_CH = 128


_SLAB = 1


def _make_deg_sweep(n_slabs):
  mesh = plsc.VectorSubcoreMesh(core_axis_name="c", subcore_axis_name="s")

  @functools.partial(
      pl.kernel,
      mesh=mesh,
      compiler_params=pltpu.CompilerParams(use_tc_tiling_on_sc=False),
      out_type=jax.ShapeDtypeStruct((2, _N_PAD, 16), jnp.float32),
      scratch_types=[
          pltpu.VMEM_SHARED((_N_PAD, 16), jnp.float32),
          pltpu.VMEM((n_slabs, _SLAB * _CH), jnp.int32),
          pltpu.VMEM((_CH, 16), jnp.float32),
          pltpu.VMEM((_CH, 16), jnp.float32),
      ],
  )
  def dsweep(dst_hbm, zeros16_hbm, ones16_hbm, dega_hbm,
             accd_sh, didx_v, ones_v, zbuf_v):
    cid = lax.axis_index("c")
    sid = lax.axis_index("s")
    wid = cid * 16 + sid
    row0 = sid * _ROWS_PER_TILE

    pltpu.sync_copy(zeros16_hbm, zbuf_v)
    for b in range(_ROWS_PER_TILE // _CH):
      pltpu.sync_copy(zbuf_v, accd_sh.at[pl.ds(row0 + b * _CH, _CH)])
    pltpu.sync_copy(ones16_hbm, ones_v)
    pltpu.sync_copy(dst_hbm.at[wid], didx_v)
    plsc.subcore_barrier()

    def body(j, carry):
      pltpu.sync_copy(ones_v, accd_sh.at[didx_v.at[j]], add=True)
      return carry

    lax.fori_loop(0, n_slabs, body, 0)
    plsc.subcore_barrier()

    for b in range(_ROWS_PER_TILE // _CH):
      pltpu.sync_copy(accd_sh.at[pl.ds(row0 + b * _CH, _CH)], zbuf_v)
      pltpu.sync_copy(zbuf_v, dega_hbm.at[cid, pl.ds(row0 + b * _CH, _CH)])

  return dsweep


def _make_prop_sweep(n_slabs):
  mesh = plsc.VectorSubcoreMesh(core_axis_name="c", subcore_axis_name="s")

  @functools.partial(
      pl.kernel,
      mesh=mesh,
      compiler_params=pltpu.CompilerParams(use_tc_tiling_on_sc=False),
      out_type=jax.ShapeDtypeStruct((2, _N_PAD, _C), jnp.float32),
      scratch_types=[
          pltpu.VMEM_SHARED((_N_PAD, _C), jnp.float32),
          pltpu.VMEM_SHARED((_N_PAD, _C), jnp.float32),
          pltpu.VMEM((n_slabs, _SLAB * _CH), jnp.int32),
          pltpu.VMEM((n_slabs, _SLAB * _CH), jnp.int32),
          pltpu.VMEM((_SLAB * _CH, _C), jnp.float32),
          pltpu.VMEM((_CH, _C), jnp.float32),
          pltpu.VMEM((_CH, _C), jnp.float32),
          pltpu.VMEM((_CH, 16), jnp.float32),
          pltpu.SemaphoreType.DMA,
          pltpu.SemaphoreType.DMA,
      ],
  )
  def msweep(agg_in_hbm, c_hbm, bh_hbm, src_hbm, dst_hbm, zeros_hbm,
             agg_hbm, acc_sh, u_sh, sidx_v, didx_v, rows_v,
             a0_v, a1_v, c_v, sem, isem):
    cid = lax.axis_index("c")
    sid = lax.axis_index("s")
    wid = cid * 16 + sid
    row0 = sid * _ROWS_PER_TILE

    bg = [pltpu.make_async_copy(src_hbm.at[wid], sidx_v, isem),
          pltpu.make_async_copy(dst_hbm.at[wid], didx_v, isem)]
    for g in bg:
      g.start()
    pltpu.sync_copy(zeros_hbm, rows_v)
    for b in range(_ROWS_PER_TILE // _CH):
      pltpu.sync_copy(rows_v, acc_sh.at[pl.ds(row0 + b * _CH, _CH)])

    for b in range(_ROWS_PER_TILE // _CH):
      r0 = row0 + b * _CH
      g0 = pltpu.make_async_copy(agg_in_hbm.at[0, pl.ds(r0, _CH)], a0_v, sem)
      g1 = pltpu.make_async_copy(agg_in_hbm.at[1, pl.ds(r0, _CH)], a1_v, sem)
      g2 = pltpu.make_async_copy(c_hbm.at[pl.ds(r0, _CH), pl.ds(0, 16)],
                                 c_v, sem)
      g3 = pltpu.make_async_copy(bh_hbm.at[pl.ds(r0, _CH)], rows_v, sem)
      g0.start()
      g1.start()
      g2.start()
      g3.start()
      g0.wait()
      g1.wait()
      g2.wait()
      g3.wait()

      def rowbody(i, carry):
        for r in (2 * i, 2 * i + 1):
          cc = c_v[r, pl.ds(0, 16)]
          for cg in range(_C // 16):
            s = pl.ds(cg * 16, 16)
            a0_v[r, s] = cc * (a0_v[r, s] + a1_v[r, s]) + rows_v[r, s]
        return carry

      lax.fori_loop(0, _CH // 2, rowbody, 0)
      pltpu.sync_copy(a0_v, u_sh.at[pl.ds(r0, _CH)])
    for g in bg:
      g.wait()
    plsc.subcore_barrier()

    def body(j, carry):
      pltpu.async_copy(u_sh.at[sidx_v.at[j]], rows_v, sem).wait()
      pltpu.sync_copy(rows_v, acc_sh.at[didx_v.at[j]], add=True)
      return carry

    lax.fori_loop(0, n_slabs, body, 0)
    plsc.subcore_barrier()

    for b in range(_ROWS_PER_TILE // _CH):
      pltpu.sync_copy(acc_sh.at[pl.ds(row0 + b * _CH, _CH)], rows_v)
      pltpu.sync_copy(rows_v, agg_hbm.at[cid, pl.ds(row0 + b * _CH, _CH)])

  return msweep


def _prep_body(x_ref, w1_ref, b1_ref, w2_ref, b2_ref, dega_ref,
               aggi_ref, c_ref, bh_ref, sd_ref):
  deg = dega_ref[0, :, 0:1] + dega_ref[1, :, 0:1]
  rows = lax.broadcasted_iota(jnp.int32, (_N_PAD, 1), 0)
  mask = rows < _N
  dinv = jnp.where(mask, lax.rsqrt(jnp.maximum(deg, 1e-12)), 0.0)
  h = jax.nn.relu(
      jnp.dot(x_ref[...], w1_ref[...], preferred_element_type=jnp.float32)
      + b1_ref[...])
  h = jnp.dot(h, w2_ref[...], preferred_element_type=jnp.float32) + b2_ref[...]
  sd = jnp.where(mask, 1.0 / jnp.where(mask, dinv, 1.0), 0.0)
  aggi_ref[0] = jnp.broadcast_to(sd, (_N_PAD, _C)) * h
  aggi_ref[1] = jnp.zeros((_N_PAD, _C), jnp.float32)
  c_ref[...] = jnp.broadcast_to((1.0 - _ALPHA) * dinv * dinv, (_N_PAD, _C))
  bh_ref[...] = _ALPHA * jnp.broadcast_to(dinv, (_N_PAD, _C)) * h
  sd_ref[...] = jnp.broadcast_to(sd, (_N_PAD, _C))


_prep_call = pl.pallas_call(
    _prep_body,
    out_shape=[
        jax.ShapeDtypeStruct((2, _N_PAD, _C), jnp.float32),
        jax.ShapeDtypeStruct((_N_PAD, _C), jnp.float32),
        jax.ShapeDtypeStruct((_N_PAD, _C), jnp.float32),
        jax.ShapeDtypeStruct((_N_PAD, _C), jnp.float32),
    ],
)


def _final_body(agg_ref, c_ref, bh_ref, sd_ref, out_ref):
  u = c_ref[...] * (agg_ref[0] + agg_ref[1]) + bh_ref[...]
  z = (u * sd_ref[...])[:_N]
  m = jnp.max(z, axis=1, keepdims=True)
  shifted = z - m
  out_ref[...] = shifted - jnp.log(
      jnp.sum(jnp.exp(shifted), axis=1, keepdims=True))


_final_call = pl.pallas_call(
    _final_body,
    out_shape=jax.ShapeDtypeStruct((_N, _C), jnp.float32),
)


def kernel(x, edge_index, W1, b1, W2, b2):
  e = edge_index.shape[1]
  e_full = e + _N
  n_slabs = -(-e_full // (_N_TILES * _SLAB * _CH))
  e_pad = _N_TILES * n_slabs * _SLAB * _CH

  src = edge_index[0]
  dst = edge_index[1]
  loop = jnp.arange(_N, dtype=jnp.int32)
  pad = jnp.full((e_pad - e_full,), _N, dtype=jnp.int32)
  src_w = jnp.concatenate([src, loop, pad]).reshape(
      _N_TILES, n_slabs, _SLAB * _CH)
  dst_w = jnp.concatenate([dst, loop, pad]).reshape(
      _N_TILES, n_slabs, _SLAB * _CH)

  x_pad = jnp.pad(x, ((0, _N_PAD - _N), (0, 0)))
  zeros_tile = jnp.zeros((_CH, _C), jnp.float32)
  zeros16 = jnp.zeros((_CH, 16), jnp.float32)
  ones16 = jnp.ones((_CH, 16), jnp.float32)

  msweep = _make_prop_sweep(n_slabs)
  dsweep = _make_deg_sweep(n_slabs)

  dega = dsweep(dst_w, zeros16, ones16)
  agg, c, bh, sd = _prep_call(x_pad, W1, b1.reshape(1, _HID), W2,
                              b2.reshape(1, _C), dega)
  for _ in range(_K):
    agg = msweep(agg, c, bh, src_w, dst_w, zeros_tile)
  return _final_call(agg, c, bh, sd)
